# baseline (speedup 1.0000x reference)
import jax
import jax.numpy as jnp
from jax.experimental import pallas as pl
from jax.experimental.pallas import tpu as pltpu

_B, _S, _N, _D_IN = 2, 4, 1024, 6
_H, _K, _L, _ND = 128, 16, 3, 3
_NG, _KG = 128, 8
_BIG_NEG = -1e30


def _topk_rem(pts, posT):
    pos = pts[:, :_ND]
    d0 = pos[:, 0:1] - posT[0:1, :]
    d1 = pos[:, 1:2] - posT[1:2, :]
    d2 = pos[:, 2:3] - posT[2:3, :]
    dist = (d0 * d0 + d1 * d1) + d2 * d2

    c_iota = jax.lax.broadcasted_iota(jnp.int32, (_N, _NG), 1)

    vs = [dist[:, _NG * k:_NG * (k + 1)] for k in range(_KG)]
    ks = [jnp.full((_N, _NG), k, jnp.int32) for k in range(_KG)]
    net = [(0, 1), (2, 3), (4, 5), (6, 7), (0, 2), (1, 3), (4, 6), (5, 7),
           (1, 2), (5, 6), (0, 4), (1, 5), (2, 6), (3, 7), (2, 4), (3, 5),
           (1, 2), (3, 4), (5, 6)]
    for n_ce, (a, b) in enumerate(net):
        if n_ce < 4:
            pred = vs[a] <= vs[b]
        else:
            pred = (vs[a] < vs[b]) | ((vs[a] == vs[b]) & (ks[a] < ks[b]))
        vs[a], vs[b] = (jnp.where(pred, vs[a], vs[b]),
                        jnp.where(pred, vs[b], vs[a]))
        ks[a], ks[b] = (jnp.where(pred, ks[a], ks[b]),
                        jnp.where(pred, ks[b], ks[a]))

    kpack = ks[0]
    for t in range(1, _KG):
        kpack = kpack | jnp.left_shift(ks[t], 3 * t)

    rem = jnp.zeros((_N, _NG), jnp.int32)
    for _ in range(_K):
        m = jnp.min(vs[0], axis=1, keepdims=True)
        k0 = jnp.bitwise_and(kpack, 7)
        jkey = jnp.where(vs[0] == m, k0 * _NG + c_iota, 2 * _N)
        jsel = jnp.min(jkey, axis=1, keepdims=True)
        sel = jkey == jsel
        rem = jnp.where(sel, rem | jnp.left_shift(1, k0), rem)
        for t in range(_KG - 1):
            vs[t] = jnp.where(sel, vs[t + 1], vs[t])
        vs[_KG - 1] = jnp.where(sel, jnp.inf, vs[_KG - 1])
        kpack = jnp.where(sel, jnp.right_shift(kpack, 3), kpack)
    return rem


def _fused_kernel(pts_t_ref, posT_t_ref, pts_l_ref, posT_l_ref,
                  Win_ref, bin_ref,
                  Wq_ref, bq_ref, Wk_ref, bk_ref, Wv_ref, bv_ref,
                  Wp_ref, bp_ref, Wpd_ref, bpd_ref, Wa_ref, ba_ref,
                  Wo_ref, bo_ref, lns_ref, lnb_ref, out_ref,
                  rem0_scr, rem1_scr):
    i = pl.program_id(0)
    slot = jnp.bitwise_and(i, 1)

    rem_prev = jnp.where(slot == 1, rem0_scr[...], rem1_scr[...])
    mask = jnp.concatenate(
        [jnp.where(jnp.bitwise_and(jnp.right_shift(rem_prev, k), 1) == 1,
                   0.0, _BIG_NEG) for k in range(_KG)], axis=1)

    pts = pts_l_ref[0]
    posT = posT_l_ref[0]
    pos = pts[:, :_ND]
    x = jnp.dot(pts, Win_ref[...],
                preferred_element_type=jnp.float32) + bin_ref[...]
    inv_scale = 1.0 / jnp.sqrt(jnp.float32(_H))

    for l in range(_L):
        q = jnp.dot(x, Wq_ref[l], preferred_element_type=jnp.float32) + bq_ref[l]
        k = jnp.dot(x, Wk_ref[l], preferred_element_type=jnp.float32) + bk_ref[l]
        v = jnp.dot(x, Wv_ref[l], preferred_element_type=jnp.float32) + bv_ref[l]
        pe = jnp.dot(pos, Wp_ref[l], preferred_element_type=jnp.float32) + bp_ref[l]
        qe = q + pe
        qw = qe * (Wa_ref[l] * inv_scale)
        kpe = k + pe

        logits = jax.lax.dot_general(
            qw, kpe, (((1,), (1,)), ((), ())),
            preferred_element_type=jnp.float32)
        u = jax.lax.dot_general(
            qw, Wpd_ref[l], (((1,), (1,)), ((), ())),
            preferred_element_type=jnp.float32)
        posdot = (u[:, 0:1] * posT[0:1, :]
                  + u[:, 1:2] * posT[1:2, :]
                  + u[:, 2:3] * posT[2:3, :])
        rowdot = jnp.sum(u * pos, axis=1, keepdims=True)
        c = jax.lax.dot_general(
            qw, bpd_ref[l], (((1,), (1,)), ((), ())),
            preferred_element_type=jnp.float32)
        logits = logits + posdot + (c - rowdot + ba_ref[l]) + mask

        mx = jnp.max(logits, axis=1, keepdims=True)
        e = jnp.exp(logits - mx)
        aw = e / jnp.sum(e, axis=1, keepdims=True)

        out = jnp.dot(aw, v, preferred_element_type=jnp.float32)
        out = jnp.dot(out, Wo_ref[l], preferred_element_type=jnp.float32) + bo_ref[l]
        out = jax.nn.gelu(out)
        x = x + out

        m = jnp.mean(x, axis=1, keepdims=True)
        xc = x - m
        var = jnp.mean(xc * xc, axis=1, keepdims=True)
        x = xc / jnp.sqrt(var + 1e-6) * lns_ref[l] + lnb_ref[l]

    out_ref[0] = jnp.max(x, axis=0, keepdims=True)

    rem = _topk_rem(pts_t_ref[0], posT_t_ref[0])

    @pl.when(slot == 0)
    def _():
        rem0_scr[...] = rem

    @pl.when(slot == 1)
    def _():
        rem1_scr[...] = rem


def _enc_kernel(x_ref, Wk_ref, bk_ref, Wq_ref, bq_ref, Wv_ref, bv_ref,
                W1_ref, b1_ref, W2_ref, b2_ref, lns_ref, lnb_ref, out_ref):
    inv_scale = 1.0 / jnp.sqrt(jnp.float32(_H))
    for b in range(_B):
        x = x_ref[b]
        k = jnp.dot(x, Wk_ref[...], preferred_element_type=jnp.float32) + bk_ref[...]
        q = jnp.dot(x, Wq_ref[...], preferred_element_type=jnp.float32) + bq_ref[...]
        v = jnp.dot(x, Wv_ref[...], preferred_element_type=jnp.float32) + bv_ref[...]
        aw = jax.lax.dot_general(
            q, k, (((1,), (1,)), ((), ())),
            preferred_element_type=jnp.float32) * inv_scale
        mx = jnp.max(aw, axis=1, keepdims=True)
        e = jnp.exp(aw - mx)
        aw = e / jnp.sum(e, axis=1, keepdims=True)
        o = jnp.dot(aw, v, preferred_element_type=jnp.float32)
        o = jnp.dot(o, W1_ref[...], preferred_element_type=jnp.float32) + b1_ref[...]
        o = jax.nn.gelu(o)
        o = jnp.dot(o, W2_ref[...], preferred_element_type=jnp.float32) + b2_ref[...]
        x = x + o
        m = jnp.mean(x, axis=1, keepdims=True)
        xc = x - m
        var = jnp.mean(xc * xc, axis=1, keepdims=True)
        x = xc / jnp.sqrt(var + 1e-6) * lns_ref[...] + lnb_ref[...]
        out_ref[b, :] = jnp.max(x, axis=0)


def _full(shape):
    nd = len(shape)
    return pl.BlockSpec(shape, lambda i, _nd=nd: (0,) * _nd)


def kernel(points, params):
    p = params
    pts = points.reshape(_B * _S, _N, _D_IN)
    posT = jnp.transpose(pts[:, :, :_ND], (0, 2, 1))

    def stk(nm):
        return jnp.stack([p[f'l{i}_{nm}'] for i in range(_L)])

    Win = p['W_in']
    bin_ = p['b_in'][None, :]
    Wq, Wk, Wv, Wo = stk('Wq'), stk('Wk'), stk('Wv'), stk('Wo')
    Wp, Wpd = stk('Wp'), stk('Wpd')
    bq = stk('bq')[:, None, :]
    bk = stk('bk')[:, None, :]
    bv = stk('bv')[:, None, :]
    bp = stk('bp')[:, None, :]
    bpd = stk('bpd')[:, None, :]
    bo = stk('bo')[:, None, :]
    Wa = jnp.stack([p[f'l{i}_Wa'][:, 0] for i in range(_L)])[:, None, :]
    ba = stk('ba')[:, :, None]
    lns = stk('ln_scale')[:, None, :]
    lnb = stk('ln_bias')[:, None, :]

    n_c = _B * _S
    topk_map = lambda i: (jnp.minimum(i, n_c - 1), 0, 0)
    layer_map = lambda i: (jnp.maximum(i - 1, 0), 0, 0)

    x8 = pl.pallas_call(
        _fused_kernel,
        grid=(n_c + 1,),
        in_specs=[
            pl.BlockSpec((1, _N, _D_IN), topk_map),
            pl.BlockSpec((1, _ND, _N), topk_map),
            pl.BlockSpec((1, _N, _D_IN), layer_map),
            pl.BlockSpec((1, _ND, _N), layer_map),
            _full(Win.shape), _full(bin_.shape),
            _full(Wq.shape), _full(bq.shape),
            _full(Wk.shape), _full(bk.shape),
            _full(Wv.shape), _full(bv.shape),
            _full(Wp.shape), _full(bp.shape),
            _full(Wpd.shape), _full(bpd.shape),
            _full(Wa.shape), _full(ba.shape),
            _full(Wo.shape), _full(bo.shape),
            _full(lns.shape), _full(lnb.shape),
        ],
        out_specs=pl.BlockSpec((1, 1, _H), layer_map),
        out_shape=jax.ShapeDtypeStruct((n_c, 1, _H), jnp.float32),
        scratch_shapes=[pltpu.VMEM((_N, _NG), jnp.int32),
                        pltpu.VMEM((_N, _NG), jnp.int32)],
        compiler_params=pltpu.CompilerParams(
            dimension_semantics=("arbitrary",)),
    )(pts, posT, pts, posT, Win, bin_, Wq, bq, Wk, bk, Wv, bv,
      Wp, bp, Wpd, bpd, Wa, ba, Wo, bo, lns, lnb)

    xs = x8.reshape(_B, _S, _H)
    out = pl.pallas_call(
        _enc_kernel,
        out_shape=jax.ShapeDtypeStruct((_B, _H), jnp.float32),
    )(xs, p['enc_Wk'], p['enc_bk'][None, :], p['enc_Wq'], p['enc_bq'][None, :],
      p['enc_Wv'], p['enc_bv'][None, :], p['enc_W1'], p['enc_b1'][None, :],
      p['enc_W2'], p['enc_b2'][None, :],
      p['lnf_scale'][None, :], p['lnf_bias'][None, :])
    return out


# speedup vs baseline: 1.2594x; 1.2594x over previous
import jax
import jax.numpy as jnp
from jax.experimental import pallas as pl
from jax.experimental.pallas import tpu as pltpu

_B, _S, _N, _D_IN = 2, 4, 1024, 6
_H, _K, _L, _ND = 128, 16, 3, 3
_BIG_NEG = -1e30


def _layers_kernel(pts_ref, posT_ref, Win_ref, bin_ref,
                   Wq_ref, bq_ref, Wk_ref, bk_ref, Wv_ref, bv_ref,
                   Wp_ref, bp_ref, Wpd_ref, bpd_ref, Wa_ref, ba_ref,
                   Wo_ref, bo_ref, lns_ref, lnb_ref, out_ref):
    pts = pts_ref[0]
    posT = posT_ref[0]
    pos = pts[:, :_ND]

    d0 = pos[:, 0:1] - posT[0:1, :]
    d1 = pos[:, 1:2] - posT[1:2, :]
    d2 = pos[:, 2:3] - posT[2:3, :]
    dist = (d0 * d0 + d1 * d1) + d2 * d2

    _NG, _KG = 128, 8
    r_iota = jax.lax.broadcasted_iota(jnp.int32, (_NG, _N), 0)

    vs = [dist[_NG * k:_NG * (k + 1), :] for k in range(_KG)]
    ks = [jnp.full((_NG, _N), k, jnp.int32) for k in range(_KG)]
    _NET = [(0, 1), (2, 3), (4, 5), (6, 7), (0, 2), (1, 3), (4, 6), (5, 7),
            (1, 2), (5, 6), (0, 4), (1, 5), (2, 6), (3, 7), (2, 4), (3, 5),
            (1, 2), (3, 4), (5, 6)]
    for n_ce, (a, b) in enumerate(_NET):
        if n_ce < 4:
            pred = vs[a] <= vs[b]
        else:
            pred = (vs[a] < vs[b]) | ((vs[a] == vs[b]) & (ks[a] < ks[b]))
        vs[a], vs[b] = (jnp.where(pred, vs[a], vs[b]),
                        jnp.where(pred, vs[b], vs[a]))
        ks[a], ks[b] = (jnp.where(pred, ks[a], ks[b]),
                        jnp.where(pred, ks[b], ks[a]))

    kpack = ks[0]
    for t in range(1, _KG):
        kpack = kpack | jnp.left_shift(ks[t], 3 * t)

    rem = jnp.zeros((_NG, _N), jnp.int32)
    for _ in range(_K):
        m = jnp.min(vs[0], axis=0, keepdims=True)
        k0 = jnp.bitwise_and(kpack, 7)
        jkey = jnp.where(vs[0] == m, k0 * _NG + r_iota, 2 * _N)
        jsel = jnp.min(jkey, axis=0, keepdims=True)
        sel = jkey == jsel
        rem = jnp.where(sel, rem | jnp.left_shift(1, k0), rem)
        for t in range(_KG - 1):
            vs[t] = jnp.where(sel, vs[t + 1], vs[t])
        vs[_KG - 1] = jnp.where(sel, jnp.inf, vs[_KG - 1])
        kpack = jnp.where(sel, jnp.right_shift(kpack, 3), kpack)
    maskT = jnp.concatenate(
        [jnp.where(jnp.bitwise_and(jnp.right_shift(rem, k), 1) == 1,
                   0.0, _BIG_NEG) for k in range(_KG)], axis=0)

    x = jnp.dot(pts, Win_ref[...],
                preferred_element_type=jnp.float32) + bin_ref[...]
    inv_scale = 1.0 / jnp.sqrt(jnp.float32(_H))

    for l in range(_L):
        q = jnp.dot(x, Wq_ref[l], preferred_element_type=jnp.float32) + bq_ref[l]
        k = jnp.dot(x, Wk_ref[l], preferred_element_type=jnp.float32) + bk_ref[l]
        v = jnp.dot(x, Wv_ref[l], preferred_element_type=jnp.float32) + bv_ref[l]
        pe = jnp.dot(pos, Wp_ref[l], preferred_element_type=jnp.float32) + bp_ref[l]
        qe = q + pe
        qw = qe * (Wa_ref[l] * inv_scale)
        kpe = k + pe

        logitsT = jax.lax.dot_general(
            kpe, qw, (((1,), (1,)), ((), ())),
            preferred_element_type=jnp.float32)
        uT = jax.lax.dot_general(
            Wpd_ref[l], qw, (((1,), (1,)), ((), ())),
            preferred_element_type=jnp.float32)
        posdotT = (pos[:, 0:1] * uT[0:1, :]
                   + pos[:, 1:2] * uT[1:2, :]
                   + pos[:, 2:3] * uT[2:3, :])
        rowdotT = jnp.sum(uT * posT, axis=0, keepdims=True)
        cT = jax.lax.dot_general(
            bpd_ref[l], qw, (((1,), (1,)), ((), ())),
            preferred_element_type=jnp.float32)
        logitsT = logitsT + posdotT + (cT - rowdotT + ba_ref[l]) + maskT

        mx = jnp.max(logitsT, axis=0, keepdims=True)
        e = jnp.exp(logitsT - mx)
        awT = e / jnp.sum(e, axis=0, keepdims=True)

        out = jax.lax.dot_general(
            awT, v, (((0,), (0,)), ((), ())),
            preferred_element_type=jnp.float32)
        out = jnp.dot(out, Wo_ref[l], preferred_element_type=jnp.float32) + bo_ref[l]
        out = jax.nn.gelu(out)
        x = x + out

        m = jnp.mean(x, axis=1, keepdims=True)
        xc = x - m
        var = jnp.mean(xc * xc, axis=1, keepdims=True)
        x = xc / jnp.sqrt(var + 1e-6) * lns_ref[l] + lnb_ref[l]

    out_ref[0] = jnp.max(x, axis=0, keepdims=True)


def _enc_kernel(x_ref, Wk_ref, bk_ref, Wq_ref, bq_ref, Wv_ref, bv_ref,
                W1_ref, b1_ref, W2_ref, b2_ref, lns_ref, lnb_ref, out_ref):
    inv_scale = 1.0 / jnp.sqrt(jnp.float32(_H))
    for b in range(_B):
        x = x_ref[b]
        k = jnp.dot(x, Wk_ref[...], preferred_element_type=jnp.float32) + bk_ref[...]
        q = jnp.dot(x, Wq_ref[...], preferred_element_type=jnp.float32) + bq_ref[...]
        v = jnp.dot(x, Wv_ref[...], preferred_element_type=jnp.float32) + bv_ref[...]
        aw = jax.lax.dot_general(
            q, k, (((1,), (1,)), ((), ())),
            preferred_element_type=jnp.float32) * inv_scale
        mx = jnp.max(aw, axis=1, keepdims=True)
        e = jnp.exp(aw - mx)
        aw = e / jnp.sum(e, axis=1, keepdims=True)
        o = jnp.dot(aw, v, preferred_element_type=jnp.float32)
        o = jnp.dot(o, W1_ref[...], preferred_element_type=jnp.float32) + b1_ref[...]
        o = jax.nn.gelu(o)
        o = jnp.dot(o, W2_ref[...], preferred_element_type=jnp.float32) + b2_ref[...]
        x = x + o
        m = jnp.mean(x, axis=1, keepdims=True)
        xc = x - m
        var = jnp.mean(xc * xc, axis=1, keepdims=True)
        x = xc / jnp.sqrt(var + 1e-6) * lns_ref[...] + lnb_ref[...]
        out_ref[b, :] = jnp.max(x, axis=0)


def _full(shape):
    nd = len(shape)
    return pl.BlockSpec(shape, lambda i, _nd=nd: (0,) * _nd)


def kernel(points, params):
    p = params
    pts = points.reshape(_B * _S, _N, _D_IN)
    posT = jnp.transpose(pts[:, :, :_ND], (0, 2, 1))

    def stk(nm):
        return jnp.stack([p[f'l{i}_{nm}'] for i in range(_L)])

    Win = p['W_in']
    bin_ = p['b_in'][None, :]
    Wq, Wk, Wv, Wo = stk('Wq'), stk('Wk'), stk('Wv'), stk('Wo')
    Wp, Wpd = stk('Wp'), stk('Wpd')
    bq = stk('bq')[:, None, :]
    bk = stk('bk')[:, None, :]
    bv = stk('bv')[:, None, :]
    bp = stk('bp')[:, None, :]
    bpd = stk('bpd')[:, None, :]
    bo = stk('bo')[:, None, :]
    Wa = jnp.stack([p[f'l{i}_Wa'][:, 0] for i in range(_L)])[:, None, :]
    ba = stk('ba')[:, :, None]
    lns = stk('ln_scale')[:, None, :]
    lnb = stk('ln_bias')[:, None, :]

    x8 = pl.pallas_call(
        _layers_kernel,
        grid=(_B * _S,),
        in_specs=[
            pl.BlockSpec((1, _N, _D_IN), lambda i: (i, 0, 0)),
            pl.BlockSpec((1, _ND, _N), lambda i: (i, 0, 0)),
            _full(Win.shape), _full(bin_.shape),
            _full(Wq.shape), _full(bq.shape),
            _full(Wk.shape), _full(bk.shape),
            _full(Wv.shape), _full(bv.shape),
            _full(Wp.shape), _full(bp.shape),
            _full(Wpd.shape), _full(bpd.shape),
            _full(Wa.shape), _full(ba.shape),
            _full(Wo.shape), _full(bo.shape),
            _full(lns.shape), _full(lnb.shape),
        ],
        out_specs=pl.BlockSpec((1, 1, _H), lambda i: (i, 0, 0)),
        out_shape=jax.ShapeDtypeStruct((_B * _S, 1, _H), jnp.float32),
        compiler_params=pltpu.CompilerParams(
            dimension_semantics=("parallel",)),
    )(pts, posT, Win, bin_, Wq, bq, Wk, bk, Wv, bv,
      Wp, bp, Wpd, bpd, Wa, ba, Wo, bo, lns, lnb)

    xs = x8.reshape(_B, _S, _H)
    out = pl.pallas_call(
        _enc_kernel,
        out_shape=jax.ShapeDtypeStruct((_B, _H), jnp.float32),
    )(xs, p['enc_Wk'], p['enc_bk'][None, :], p['enc_Wq'], p['enc_bq'][None, :],
      p['enc_Wv'], p['enc_bv'][None, :], p['enc_W1'], p['enc_b1'][None, :],
      p['enc_W2'], p['enc_b2'][None, :],
      p['lnf_scale'][None, :], p['lnf_bias'][None, :])
    return out


# speedup vs baseline: 1.2691x; 1.0077x over previous
import jax
import jax.numpy as jnp
from jax.experimental import pallas as pl
from jax.experimental.pallas import tpu as pltpu

_B, _S, _N, _D_IN = 2, 4, 1024, 6
_H, _K, _L, _ND = 128, 16, 3, 3
_BIG_NEG = -1e30


def _layers_kernel(pts_ref, posT_ref, Win_ref, bin_ref,
                   Wq_ref, bq_ref, Wk_ref, bk_ref, Wv_ref, bv_ref,
                   Wp_ref, bp_ref, Wpd_ref, bpd_ref, Wa_ref, ba_ref,
                   Wo_ref, bo_ref, lns_ref, lnb_ref, out_ref):
    pts = pts_ref[0]
    posT = posT_ref[0]
    pos = pts[:, :_ND]

    d0 = pos[:, 0:1] - posT[0:1, :]
    d1 = pos[:, 1:2] - posT[1:2, :]
    d2 = pos[:, 2:3] - posT[2:3, :]
    dist = (d0 * d0 + d1 * d1) + d2 * d2

    _NG, _KG = 128, 8
    r_iota = jax.lax.broadcasted_iota(jnp.int32, (_NG, _N), 0)

    vs = [dist[_NG * k:_NG * (k + 1), :] for k in range(_KG)]
    ks = [jnp.full((_NG, _N), k, jnp.int32) for k in range(_KG)]
    _NET = [(0, 1), (2, 3), (4, 5), (6, 7), (0, 2), (1, 3), (4, 6), (5, 7),
            (1, 2), (5, 6), (0, 4), (1, 5), (2, 6), (3, 7), (2, 4), (3, 5),
            (1, 2), (3, 4), (5, 6)]
    for n_ce, (a, b) in enumerate(_NET):
        if n_ce < 4:
            pred = vs[a] <= vs[b]
        else:
            pred = (vs[a] < vs[b]) | ((vs[a] == vs[b]) & (ks[a] < ks[b]))
        vs[a], vs[b] = (jnp.where(pred, vs[a], vs[b]),
                        jnp.where(pred, vs[b], vs[a]))
        ks[a], ks[b] = (jnp.where(pred, ks[a], ks[b]),
                        jnp.where(pred, ks[b], ks[a]))

    kpack = ks[0]
    for t in range(1, _KG):
        kpack = kpack | jnp.left_shift(ks[t], 3 * t)

    rem = jnp.zeros((_NG, _N), jnp.int32)
    for _ in range(_K):
        m = jnp.min(vs[0], axis=0, keepdims=True)
        k0 = jnp.bitwise_and(kpack, 7)
        jkey = jnp.where(vs[0] == m, k0 * _NG + r_iota, 2 * _N)
        jsel = jnp.min(jkey, axis=0, keepdims=True)
        sel = jkey == jsel
        rem = jnp.where(sel, rem | jnp.left_shift(1, k0), rem)
        for t in range(_KG - 1):
            vs[t] = jnp.where(sel, vs[t + 1], vs[t])
        vs[_KG - 1] = jnp.where(sel, jnp.inf, vs[_KG - 1])
        kpack = jnp.where(sel, jnp.right_shift(kpack, 3), kpack)
    maskT = jnp.concatenate(
        [jnp.where(jnp.bitwise_and(jnp.right_shift(rem, k), 1) == 1,
                   0.0, _BIG_NEG) for k in range(_KG)], axis=0)

    x = jnp.dot(pts, Win_ref[...],
                preferred_element_type=jnp.float32) + bin_ref[...]
    inv_scale = 1.0 / jnp.sqrt(jnp.float32(_H))
    ones_n1 = jnp.ones((_N, 1), jnp.float32)

    for l in range(_L):
        q = jnp.dot(x, Wq_ref[l], preferred_element_type=jnp.float32) + bq_ref[l]
        k = jnp.dot(x, Wk_ref[l], preferred_element_type=jnp.float32) + bk_ref[l]
        v = jnp.dot(x, Wv_ref[l], preferred_element_type=jnp.float32) + bv_ref[l]
        pe = jnp.dot(pos, Wp_ref[l], preferred_element_type=jnp.float32) + bp_ref[l]
        qe = q + pe
        qw = qe * (Wa_ref[l] * inv_scale)
        kpe = k + pe

        logitsT = jax.lax.dot_general(
            kpe, qw, (((1,), (1,)), ((), ())),
            preferred_element_type=jnp.float32)
        uT = jax.lax.dot_general(
            Wpd_ref[l], qw, (((1,), (1,)), ((), ())),
            preferred_element_type=jnp.float32)
        posdotT = jax.lax.dot_general(
            pos, uT, (((1,), (0,)), ((), ())),
            preferred_element_type=jnp.float32)
        logitsT = (logitsT + posdotT) + maskT

        mx = jnp.max(logitsT, axis=0, keepdims=True)
        e = jnp.exp(logitsT - mx)
        denomT = jax.lax.dot_general(
            e, ones_n1, (((0,), (0,)), ((), ())),
            preferred_element_type=jnp.float32)
        out = jax.lax.dot_general(
            e, v, (((0,), (0,)), ((), ())),
            preferred_element_type=jnp.float32)
        out = out * (1.0 / denomT)
        out = jnp.dot(out, Wo_ref[l], preferred_element_type=jnp.float32) + bo_ref[l]
        out = jax.nn.gelu(out)
        x = x + out

        m = jnp.mean(x, axis=1, keepdims=True)
        xc = x - m
        var = jnp.mean(xc * xc, axis=1, keepdims=True)
        x = xc / jnp.sqrt(var + 1e-6) * lns_ref[l] + lnb_ref[l]

    out_ref[0] = jnp.max(x, axis=0, keepdims=True)


def _enc_kernel(x_ref, Wk_ref, bk_ref, Wq_ref, bq_ref, Wv_ref, bv_ref,
                W1_ref, b1_ref, W2_ref, b2_ref, lns_ref, lnb_ref, out_ref):
    inv_scale = 1.0 / jnp.sqrt(jnp.float32(_H))
    for b in range(_B):
        x = x_ref[b]
        k = jnp.dot(x, Wk_ref[...], preferred_element_type=jnp.float32) + bk_ref[...]
        q = jnp.dot(x, Wq_ref[...], preferred_element_type=jnp.float32) + bq_ref[...]
        v = jnp.dot(x, Wv_ref[...], preferred_element_type=jnp.float32) + bv_ref[...]
        aw = jax.lax.dot_general(
            q, k, (((1,), (1,)), ((), ())),
            preferred_element_type=jnp.float32) * inv_scale
        mx = jnp.max(aw, axis=1, keepdims=True)
        e = jnp.exp(aw - mx)
        aw = e / jnp.sum(e, axis=1, keepdims=True)
        o = jnp.dot(aw, v, preferred_element_type=jnp.float32)
        o = jnp.dot(o, W1_ref[...], preferred_element_type=jnp.float32) + b1_ref[...]
        o = jax.nn.gelu(o)
        o = jnp.dot(o, W2_ref[...], preferred_element_type=jnp.float32) + b2_ref[...]
        x = x + o
        m = jnp.mean(x, axis=1, keepdims=True)
        xc = x - m
        var = jnp.mean(xc * xc, axis=1, keepdims=True)
        x = xc / jnp.sqrt(var + 1e-6) * lns_ref[...] + lnb_ref[...]
        out_ref[b, :] = jnp.max(x, axis=0)


def _full(shape):
    nd = len(shape)
    return pl.BlockSpec(shape, lambda i, _nd=nd: (0,) * _nd)


def kernel(points, params):
    p = params
    pts = points.reshape(_B * _S, _N, _D_IN)
    posT = jnp.transpose(pts[:, :, :_ND], (0, 2, 1))

    def stk(nm):
        return jnp.stack([p[f'l{i}_{nm}'] for i in range(_L)])

    Win = p['W_in']
    bin_ = p['b_in'][None, :]
    Wq, Wk, Wv, Wo = stk('Wq'), stk('Wk'), stk('Wv'), stk('Wo')
    Wp, Wpd = stk('Wp'), stk('Wpd')
    bq = stk('bq')[:, None, :]
    bk = stk('bk')[:, None, :]
    bv = stk('bv')[:, None, :]
    bp = stk('bp')[:, None, :]
    bpd = stk('bpd')[:, None, :]
    bo = stk('bo')[:, None, :]
    Wa = jnp.stack([p[f'l{i}_Wa'][:, 0] for i in range(_L)])[:, None, :]
    ba = stk('ba')[:, :, None]
    lns = stk('ln_scale')[:, None, :]
    lnb = stk('ln_bias')[:, None, :]

    x8 = pl.pallas_call(
        _layers_kernel,
        grid=(_B * _S,),
        in_specs=[
            pl.BlockSpec((1, _N, _D_IN), lambda i: (i, 0, 0)),
            pl.BlockSpec((1, _ND, _N), lambda i: (i, 0, 0)),
            _full(Win.shape), _full(bin_.shape),
            _full(Wq.shape), _full(bq.shape),
            _full(Wk.shape), _full(bk.shape),
            _full(Wv.shape), _full(bv.shape),
            _full(Wp.shape), _full(bp.shape),
            _full(Wpd.shape), _full(bpd.shape),
            _full(Wa.shape), _full(ba.shape),
            _full(Wo.shape), _full(bo.shape),
            _full(lns.shape), _full(lnb.shape),
        ],
        out_specs=pl.BlockSpec((1, 1, _H), lambda i: (i, 0, 0)),
        out_shape=jax.ShapeDtypeStruct((_B * _S, 1, _H), jnp.float32),
        compiler_params=pltpu.CompilerParams(
            dimension_semantics=("parallel",)),
    )(pts, posT, Win, bin_, Wq, bq, Wk, bk, Wv, bv,
      Wp, bp, Wpd, bpd, Wa, ba, Wo, bo, lns, lnb)

    xs = x8.reshape(_B, _S, _H)
    out = pl.pallas_call(
        _enc_kernel,
        out_shape=jax.ShapeDtypeStruct((_B, _H), jnp.float32),
    )(xs, p['enc_Wk'], p['enc_bk'][None, :], p['enc_Wq'], p['enc_bq'][None, :],
      p['enc_Wv'], p['enc_bv'][None, :], p['enc_W1'], p['enc_b1'][None, :],
      p['enc_W2'], p['enc_b2'][None, :],
      p['lnf_scale'][None, :], p['lnf_bias'][None, :])
    return out


# speedup vs baseline: 1.2721x; 1.0024x over previous
import jax
import jax.numpy as jnp
from jax.experimental import pallas as pl
from jax.experimental.pallas import tpu as pltpu

_B, _S, _N, _D_IN = 2, 4, 1024, 6
_H, _K, _L, _ND = 128, 16, 3, 3
_BIG_NEG = -1e30


def _layers_kernel(pts_ref, posT_ref, Win_ref, bin_ref,
                   Wq_ref, bq_ref, Wk_ref, bk_ref, Wv_ref, bv_ref,
                   Wp_ref, bp_ref, Wpd_ref, bpd_ref, Wa_ref, ba_ref,
                   Wo_ref, bo_ref, lns_ref, lnb_ref,
                   eWk_ref, ebk_ref, eWq_ref, ebq_ref, eWv_ref, ebv_ref,
                   eW1_ref, eb1_ref, eW2_ref, eb2_ref, elns_ref, elnb_ref,
                   out_ref, xacc_ref):
    pts = pts_ref[0]
    posT = posT_ref[0]
    pos = pts[:, :_ND]

    d0 = pos[:, 0:1] - posT[0:1, :]
    d1 = pos[:, 1:2] - posT[1:2, :]
    d2 = pos[:, 2:3] - posT[2:3, :]
    dist = (d0 * d0 + d1 * d1) + d2 * d2

    _NG, _KG = 128, 8
    r_iota = jax.lax.broadcasted_iota(jnp.int32, (_NG, _N), 0)

    vs = [dist[_NG * k:_NG * (k + 1), :] for k in range(_KG)]
    ks = [jnp.full((_NG, _N), k, jnp.int32) for k in range(_KG)]
    _NET = [(0, 1), (2, 3), (4, 5), (6, 7), (0, 2), (1, 3), (4, 6), (5, 7),
            (1, 2), (5, 6), (0, 4), (1, 5), (2, 6), (3, 7), (2, 4), (3, 5),
            (1, 2), (3, 4), (5, 6)]
    for n_ce, (a, b) in enumerate(_NET):
        if n_ce < 4:
            pred = vs[a] <= vs[b]
        else:
            pred = (vs[a] < vs[b]) | ((vs[a] == vs[b]) & (ks[a] < ks[b]))
        vs[a], vs[b] = (jnp.where(pred, vs[a], vs[b]),
                        jnp.where(pred, vs[b], vs[a]))
        ks[a], ks[b] = (jnp.where(pred, ks[a], ks[b]),
                        jnp.where(pred, ks[b], ks[a]))

    kpack = ks[0]
    for t in range(1, _KG):
        kpack = kpack | jnp.left_shift(ks[t], 3 * t)

    rem = jnp.zeros((_NG, _N), jnp.int32)
    for _ in range(_K):
        m = jnp.min(vs[0], axis=0, keepdims=True)
        k0 = jnp.bitwise_and(kpack, 7)
        jkey = jnp.where(vs[0] == m, k0 * _NG + r_iota, 2 * _N)
        jsel = jnp.min(jkey, axis=0, keepdims=True)
        sel = jkey == jsel
        rem = jnp.where(sel, rem | jnp.left_shift(1, k0), rem)
        for t in range(_KG - 1):
            vs[t] = jnp.where(sel, vs[t + 1], vs[t])
        vs[_KG - 1] = jnp.where(sel, jnp.inf, vs[_KG - 1])
        kpack = jnp.where(sel, jnp.right_shift(kpack, 3), kpack)
    maskT = jnp.concatenate(
        [jnp.where(jnp.bitwise_and(jnp.right_shift(rem, k), 1) == 1,
                   0.0, _BIG_NEG) for k in range(_KG)], axis=0)

    x = jnp.dot(pts, Win_ref[...],
                preferred_element_type=jnp.float32) + bin_ref[...]
    inv_scale = 1.0 / jnp.sqrt(jnp.float32(_H))
    ones_n1 = jnp.ones((_N, 1), jnp.float32)

    for l in range(_L):
        q = jnp.dot(x, Wq_ref[l], preferred_element_type=jnp.float32) + bq_ref[l]
        k = jnp.dot(x, Wk_ref[l], preferred_element_type=jnp.float32) + bk_ref[l]
        v = jnp.dot(x, Wv_ref[l], preferred_element_type=jnp.float32) + bv_ref[l]
        pe = jnp.dot(pos, Wp_ref[l], preferred_element_type=jnp.float32) + bp_ref[l]
        qe = q + pe
        qw = qe * (Wa_ref[l] * inv_scale)
        kpe = k + pe

        logitsT = jax.lax.dot_general(
            kpe, qw, (((1,), (1,)), ((), ())),
            preferred_element_type=jnp.float32)
        uT = jax.lax.dot_general(
            Wpd_ref[l], qw, (((1,), (1,)), ((), ())),
            preferred_element_type=jnp.float32)
        posdotT = jax.lax.dot_general(
            pos, uT, (((1,), (0,)), ((), ())),
            preferred_element_type=jnp.float32)
        logitsT = (logitsT + posdotT) + maskT

        mx = jnp.max(logitsT, axis=0, keepdims=True)
        e = jnp.exp(logitsT - mx)
        denomT = jax.lax.dot_general(
            e, ones_n1, (((0,), (0,)), ((), ())),
            preferred_element_type=jnp.float32)
        out = jax.lax.dot_general(
            e, v, (((0,), (0,)), ((), ())),
            preferred_element_type=jnp.float32)
        out = out * (1.0 / denomT)
        out = jnp.dot(out, Wo_ref[l], preferred_element_type=jnp.float32) + bo_ref[l]
        out = jax.nn.gelu(out)
        x = x + out

        m = jnp.mean(x, axis=1, keepdims=True)
        xc = x - m
        var = jnp.mean(xc * xc, axis=1, keepdims=True)
        x = xc / jnp.sqrt(var + 1e-6) * lns_ref[l] + lnb_ref[l]

    xmax = jnp.max(x, axis=0, keepdims=True)
    i = pl.program_id(0)
    xacc_ref[pl.ds(i, 1), :] = xmax

    @pl.when(i == _B * _S - 1)
    def _():
        for b in range(_B):
            xb = xacc_ref[b * _S:(b + 1) * _S, :]
            k = jnp.dot(xb, eWk_ref[...], preferred_element_type=jnp.float32) + ebk_ref[...]
            q = jnp.dot(xb, eWq_ref[...], preferred_element_type=jnp.float32) + ebq_ref[...]
            v = jnp.dot(xb, eWv_ref[...], preferred_element_type=jnp.float32) + ebv_ref[...]
            aw = jax.lax.dot_general(
                q, k, (((1,), (1,)), ((), ())),
                preferred_element_type=jnp.float32) * inv_scale
            mx2 = jnp.max(aw, axis=1, keepdims=True)
            e2 = jnp.exp(aw - mx2)
            aw = e2 / jnp.sum(e2, axis=1, keepdims=True)
            o = jnp.dot(aw, v, preferred_element_type=jnp.float32)
            o = jnp.dot(o, eW1_ref[...], preferred_element_type=jnp.float32) + eb1_ref[...]
            o = jax.nn.gelu(o)
            o = jnp.dot(o, eW2_ref[...], preferred_element_type=jnp.float32) + eb2_ref[...]
            xe = xb + o
            m2 = jnp.mean(xe, axis=1, keepdims=True)
            xc2 = xe - m2
            var2 = jnp.mean(xc2 * xc2, axis=1, keepdims=True)
            xe = xc2 / jnp.sqrt(var2 + 1e-6) * elns_ref[...] + elnb_ref[...]
            out_ref[b, :] = jnp.max(xe, axis=0)


def _full(shape):
    nd = len(shape)
    return pl.BlockSpec(shape, lambda i, _nd=nd: (0,) * _nd)


def kernel(points, params):
    p = params
    pts = points.reshape(_B * _S, _N, _D_IN)
    posT = jnp.transpose(pts[:, :, :_ND], (0, 2, 1))

    def stk(nm):
        return jnp.stack([p[f'l{i}_{nm}'] for i in range(_L)])

    Win = p['W_in']
    bin_ = p['b_in'][None, :]
    Wq, Wk, Wv, Wo = stk('Wq'), stk('Wk'), stk('Wv'), stk('Wo')
    Wp, Wpd = stk('Wp'), stk('Wpd')
    bq = stk('bq')[:, None, :]
    bk = stk('bk')[:, None, :]
    bv = stk('bv')[:, None, :]
    bp = stk('bp')[:, None, :]
    bpd = stk('bpd')[:, None, :]
    bo = stk('bo')[:, None, :]
    Wa = jnp.stack([p[f'l{i}_Wa'][:, 0] for i in range(_L)])[:, None, :]
    ba = stk('ba')[:, :, None]
    lns = stk('ln_scale')[:, None, :]
    lnb = stk('ln_bias')[:, None, :]

    enc_args = (p['enc_Wk'], p['enc_bk'][None, :],
                p['enc_Wq'], p['enc_bq'][None, :],
                p['enc_Wv'], p['enc_bv'][None, :],
                p['enc_W1'], p['enc_b1'][None, :],
                p['enc_W2'], p['enc_b2'][None, :],
                p['lnf_scale'][None, :], p['lnf_bias'][None, :])

    out = pl.pallas_call(
        _layers_kernel,
        grid=(_B * _S,),
        in_specs=[
            pl.BlockSpec((1, _N, _D_IN), lambda i: (i, 0, 0)),
            pl.BlockSpec((1, _ND, _N), lambda i: (i, 0, 0)),
            _full(Win.shape), _full(bin_.shape),
            _full(Wq.shape), _full(bq.shape),
            _full(Wk.shape), _full(bk.shape),
            _full(Wv.shape), _full(bv.shape),
            _full(Wp.shape), _full(bp.shape),
            _full(Wpd.shape), _full(bpd.shape),
            _full(Wa.shape), _full(ba.shape),
            _full(Wo.shape), _full(bo.shape),
            _full(lns.shape), _full(lnb.shape),
        ] + [_full(a.shape) for a in enc_args],
        out_specs=pl.BlockSpec((_B, _H), lambda i: (0, 0)),
        out_shape=jax.ShapeDtypeStruct((_B, _H), jnp.float32),
        scratch_shapes=[pltpu.VMEM((_B * _S, _H), jnp.float32)],
        compiler_params=pltpu.CompilerParams(
            dimension_semantics=("arbitrary",)),
    )(pts, posT, Win, bin_, Wq, bq, Wk, bk, Wv, bv,
      Wp, bp, Wpd, bpd, Wa, ba, Wo, bo, lns, lnb, *enc_args)
    return out
